# baseline (speedup 1.0000x reference)
import functools

import jax
import jax.numpy as jnp
from jax import lax
from jax.experimental import pallas as pl
from jax.experimental.pallas import tpu as pltpu
from jax.experimental.pallas import tpu_sc as plsc

_BS = 128
_AT = 64
_EM = 2048
_HID = 128
_NF = 128
_NG = 50
_NI = 3
_CUT = 10.0
_N = _BS * _AT
_NK = _AT * _AT
_NFA = _NI * _NF
_NW = 32
_EPW = _EM // _NW
_RPS = _NK // 16
_LN2 = 0.6931471805599453
_PREC = jax.lax.Precision.DEFAULT
_XPREC = jax.lax.Precision.HIGHEST


def _ssp(x):
    return jnp.maximum(x, 0.0) + jnp.log1p(jnp.exp(-jnp.abs(x))) - _LN2


def _edge_body(ew_col, w1c, b1r, w2s, b2r, wft_ref):
    f32 = jnp.float32
    ew = ew_col[...]
    step = _CUT / (_NG - 1)
    coeff = -0.5 / step ** 2
    off = lax.broadcasted_iota(jnp.int32, (_EM, 64), 1).astype(f32) * step
    a = jnp.exp(coeff * (ew - off) ** 2)
    hid = _ssp(jnp.dot(a, w1c[...], preferred_element_type=f32,
                       precision=_PREC) + b1r[...])
    cosc = 0.5 * (jnp.cos(ew * (jnp.pi / _CUT)) + 1.0)
    for i in range(_NI):
        hi = hid[:, i * _NF:(i + 1) * _NF]
        w = jnp.dot(hi, w2s[i], preferred_element_type=f32,
                    precision=_PREC) + b2r[i]
        wft_ref[:, i * _NF:(i + 1) * _NF] = w * cosc


_KPW = _NK // _NW
_FCH = _NFA // 16


def _sc_scatter_body(wft_hbm, keys_hbm, out_hbm, keys_v, elist, rows_v, s2, sem):
    i32 = jnp.int32
    c = lax.axis_index("c")
    s = lax.axis_index("s")
    wid = s * 2 + c
    lo = wid * _KPW
    iota = lax.iota(i32, 16)
    zeros16 = jnp.zeros((16,), jnp.float32)
    pltpu.sync_copy(keys_hbm, keys_v)

    def _zero(r, carry):
        plsc.store_scatter(s2, [r * 16 + iota], zeros16)
        return carry
    lax.fori_loop(0, _KPW * _NFA // 16, _zero, 0, unroll=False)

    def _compact(i, cnt_vec):
        kv = plsc.load_gather(keys_v, [i * 16 + iota])
        m = (kv >= lo) & (kv < lo + _KPW)
        pos = cnt_vec + plsc.cumsum(m.astype(i32)) - 1
        plsc.store_scatter(elist, [pos], i * 16 + iota, mask=m)
        return cnt_vec + plsc.all_reduce_population_count(m)
    cnt_vec = lax.fori_loop(0, _EM // 16, _compact,
                            jnp.zeros((16,), i32), unroll=False)
    cnt = jnp.max(cnt_vec)

    def _accum(g, carry):
        valid = (g * 16 + iota) < cnt_vec
        e16 = plsc.load_gather(elist, [g * 16 + iota])
        e16 = jnp.where(valid, e16, 0)
        pltpu.async_copy(wft_hbm.at[e16], rows_v, sem).wait()
        for j in range(16):
            ej = plsc.load_gather(elist, [jnp.zeros((16,), i32) + g * 16 + j])
            ej = jnp.where((g * 16 + j) < cnt_vec, ej, 0)
            kj = plsc.load_gather(keys_v, [ej]) - lo
            mj = (g * 16 + j) < cnt_vec
            for cch in range(_FCH):
                val = rows_v[j, pl.ds(cch * 16, 16)]
                plsc.addupdate_scatter(
                    s2, [kj * _NFA + cch * 16 + iota], val, mask=mj)
        return carry
    lax.fori_loop(0, (cnt + 15) // 16, _accum, 0, unroll=False)

    pltpu.sync_copy(s2, out_hbm.at[pl.ds(wid * _KPW * _NFA, _KPW * _NFA)])


_sc_scatter = functools.partial(
    pl.kernel,
    mesh=plsc.VectorSubcoreMesh(core_axis_name="c", subcore_axis_name="s"),
    out_type=jax.ShapeDtypeStruct((_NK * _NFA,), jnp.float32),
    compiler_params=pltpu.CompilerParams(use_tc_tiling_on_sc=False,
                                         needs_layout_passes=False),
    scratch_types=[
        pltpu.VMEM((_EM,), jnp.int32),
        pltpu.VMEM((_EM + 16,), jnp.int32),
        pltpu.VMEM((16, _NFA), jnp.float32),
        pltpu.VMEM((_KPW * _NFA,), jnp.float32),
        pltpu.SemaphoreType.DMA,
    ],
)(_sc_scatter_body)


def _main_body(x_row, s_nmaj,
               cw1t, cw2t, cb2c, lwt, lbc,
               embt, embb, l1t, l1b, l2t, l2b,
               out_ref):
    f32 = jnp.float32
    s_all = jnp.transpose(s_nmaj[...])
    h = embt[...] * x_row[...] + embb[...]
    for i in range(_NI):
        x1 = jnp.dot(cw1t[i], h, preferred_element_type=f32, precision=_PREC)
        x1r = x1.reshape(_NF, _BS, _AT)
        sr = s_all[i * _NF:(i + 1) * _NF, :].reshape(_NF, _AT, _AT)
        aggr = lax.dot_general(x1r, sr, (((2,), (1,)), ((0,), (0,))),
                               preferred_element_type=f32, precision=_XPREC)
        agg = aggr.reshape(_NF, _N)
        x2 = jnp.dot(cw2t[i], agg, preferred_element_type=f32,
                     precision=_PREC) + cb2c[i]
        x3 = _ssp(x2)
        h = h + jnp.dot(lwt[i], x3, preferred_element_type=f32,
                        precision=_PREC) + lbc[i]
    g = jnp.dot(l1t[...], h, preferred_element_type=f32,
                precision=_PREC) + l1b[...]
    o = jnp.sum(g * l2t[...], axis=0, keepdims=True) + l2b[...]
    pmat = ((lax.broadcasted_iota(jnp.int32, (_N, _BS), 0) // _AT)
            == lax.broadcasted_iota(jnp.int32, (_N, _BS), 1)).astype(f32)
    out_ref[...] = jnp.dot(o, pmat, preferred_element_type=f32,
                           precision=_XPREC)


def kernel(sites, edge_idx, edge_weight, params):
    p = params
    blocks = p['blocks']
    f32 = jnp.float32
    x_row = sites.astype(f32).reshape(1, _N)
    ew_col = edge_weight.astype(f32).reshape(_EM, 1)
    ei = edge_idx.astype(jnp.int32)
    keys_flat = (ei[:, 0] * _AT + ei[:, 1]).reshape(_EM)
    w1c = jnp.concatenate(
        [jnp.pad(b['mlp_w1'], ((0, 64 - _NG), (0, 0))) for b in blocks], axis=1)
    b1r = jnp.concatenate([b['mlp_b1'].reshape(1, _NF) for b in blocks], axis=1)
    w2s = jnp.stack([b['mlp_w2'] for b in blocks])
    b2r = jnp.stack([b['mlp_b2'].reshape(1, _NF) for b in blocks])
    wft = pl.pallas_call(
        _edge_body,
        out_shape=jax.ShapeDtypeStruct((_EM, _NFA), f32),
        compiler_params=pltpu.CompilerParams(
            vmem_limit_bytes=100 * 1024 * 1024),
    )(ew_col, w1c, b1r, w2s, b2r)
    s_nmaj = _sc_scatter(wft, keys_flat).reshape(_NK, _NFA)
    cw1t = jnp.stack([b['conv_w1'].T for b in blocks])
    cw2t = jnp.stack([b['conv_w2'].T for b in blocks])
    cb2c = jnp.stack([b['conv_b2'].reshape(_HID, 1) for b in blocks])
    lwt = jnp.stack([b['lin_w'].T for b in blocks])
    lbc = jnp.stack([b['lin_b'].reshape(_HID, 1) for b in blocks])
    embt = p['emb_w'].T
    embb = p['emb_b'].reshape(_HID, 1)
    l1t = p['lin1_w'].T
    l1b = p['lin1_b'].reshape(_HID // 2, 1)
    l2t = p['lin2_w']
    l2b = p['lin2_b'].reshape(1, 1)
    out = pl.pallas_call(
        _main_body,
        out_shape=jax.ShapeDtypeStruct((1, _BS), f32),
        compiler_params=pltpu.CompilerParams(
            vmem_limit_bytes=100 * 1024 * 1024),
    )(x_row, s_nmaj,
      cw1t, cw2t, cb2c, lwt, lbc,
      embt, embb, l1t, l1b, l2t, l2b)
    return out.reshape(_BS, 1)
